# baseline (speedup 1.0000x reference)
import functools

import numpy as np

import jax
import jax.numpy as jnp
from jax.experimental import pallas as pl
from jax.experimental.pallas import tpu as pltpu

_K = 5
_CO1 = 16
_CO2 = 24
_CO2R = 20
_WQ = 21
_ND1 = 21 * _WQ
_ND2 = 18 * _WQ
_L16 = 22 * _WQ + 1


def _phase16_split(x):
    b, c, h, w = x.shape
    hq, wq = h // 4, w // 4
    t = x.astype(jnp.bfloat16)
    t = t.reshape(b, c, hq, 4, wq, 4).transpose(0, 3, 5, 1, 2, 4)
    t = t.reshape(b, 16 * c, hq, wq)
    t = jnp.pad(t, ((0, 0), (0, 0), (0, 1), (0, 0)))
    t = t.reshape(b, 16 * c, (hq + 1) * wq)
    return jnp.pad(t, ((0, 0), (0, 0), (0, 1)))


@functools.lru_cache(maxsize=None)
def _placement1():
    t = np.zeros((4, 4, _K * _K, 64), np.float32)
    for p in (0, 1):
        for q in (0, 1):
            for po in range(4):
                pa, pb = po // 2, po % 2
                for kh in range(_K):
                    for kw in range(_K):
                        dr, rm = divmod(2 * p + pa + kh, 4)
                        dc, cm = divmod(2 * q + pb + kw, 4)
                        slot = (dr * 2 + dc) * 16 + rm * 4 + cm
                        t[p * 2 + q, po, kh * _K + kw, slot] = 1.0
    return t


@functools.lru_cache(maxsize=None)
def _placement2():
    t = np.zeros((4, _K * _K, 36), np.float32)
    for po in range(4):
        pa, pb = po // 2, po % 2
        for kh in range(_K):
            for kw in range(_K):
                dr, p = divmod(pa + kh, 2)
                dc, q = divmod(pb + kw, 2)
                t[po, kh * _K + kw, (dr * 3 + dc) * 4 + p * 2 + q] = 1.0
    return t


def _build_w1(conv1_w, cin):
    ws = jnp.einsum("toi,qpts->qposi", conv1_w, jnp.asarray(_placement1()))
    ws = jnp.pad(ws, ((0, 0), (0, 0), (0, _CO1 - conv1_w.shape[1]),
                      (0, 0), (0, 0)))
    return ws.reshape(4 * 4 * _CO1, 64 * cin).astype(jnp.bfloat16)


def _build_w2(conv2_w):
    ws = jnp.einsum("toi,pts->posi", conv2_w, jnp.asarray(_placement2()))
    ws = jnp.pad(ws, ((0, 0), (0, _CO2 - conv2_w.shape[1]),
                      (0, 0), (0, _CO1 - conv2_w.shape[2])))
    return ws.reshape(4 * _CO2, 36 * _CO1).astype(jnp.bfloat16)


def _conv_tower_body(xq_ref, w1_ref, b1_ref, w2_ref, b2_ref, o_ref, *,
                     cin, img):
    w1 = w1_ref[...]
    b1 = b1_ref[...]
    w2 = w2_ref[...]
    b2 = b2_ref[...]
    y1cats = []
    for i in range(img):
        xcat = jnp.concatenate(
            [xq_ref[i, :, off:off + _ND1] for off in (0, 1, _WQ, _WQ + 1)],
            axis=0)
        acc = jnp.dot(w1, xcat, preferred_element_type=jnp.float32)
        y1 = []
        for pq in range(4):
            a = acc[pq * 4 * _CO1:(pq + 1) * 4 * _CO1]
            pooled = jnp.maximum(
                jnp.maximum(a[:_CO1], a[_CO1:2 * _CO1]),
                jnp.maximum(a[2 * _CO1:3 * _CO1], a[3 * _CO1:]))
            y1.append(jnp.maximum(pooled + b1, 0.0)
                      .astype(jnp.bfloat16))
        y1cats.append(jnp.concatenate(y1, axis=0))
    for i in range(img):
        xcat2 = jnp.concatenate(
            [y1cats[i][:, dr * _WQ + dc:dr * _WQ + dc + _ND2]
             for dr in range(3) for dc in range(3)],
            axis=0)
        acc2 = jnp.dot(w2, xcat2, preferred_element_type=jnp.float32)
        pooled = jnp.maximum(
            jnp.maximum(acc2[:_CO2], acc2[_CO2:2 * _CO2]),
            jnp.maximum(acc2[2 * _CO2:3 * _CO2], acc2[3 * _CO2:]))
        o_ref[i] = jnp.maximum(pooled[:_CO2R] + b2, 0.0).astype(jnp.bfloat16)


def _fc_body(x_ref, w1_ref, b1_ref, w2_ref, b2_ref, o_ref):
    h = jnp.dot(x_ref[...], w1_ref[...], preferred_element_type=jnp.float32)
    h = jnp.maximum(h + b1_ref[...], 0.0)
    z = jnp.dot(h, w2_ref[...], preferred_element_type=jnp.float32)
    z = z + b2_ref[...]
    z = z - jnp.max(z, axis=-1, keepdims=True)
    o_ref[...] = z - jnp.log(jnp.sum(jnp.exp(z), axis=-1, keepdims=True))


def _conv_tower(xq, w1s, b1, w2s, b2, *, cin, img):
    b = xq.shape[0]
    body = functools.partial(_conv_tower_body, cin=cin, img=img)
    return pl.pallas_call(
        body,
        out_shape=jax.ShapeDtypeStruct((b, _CO2R, _ND2), jnp.bfloat16),
        grid=(b // img,),
        in_specs=[
            pl.BlockSpec((img, 16 * cin, _L16), lambda i: (i, 0, 0)),
            pl.BlockSpec((4 * 4 * _CO1, 64 * cin), lambda i: (0, 0)),
            pl.BlockSpec((_CO1, 1), lambda i: (0, 0)),
            pl.BlockSpec((4 * _CO2, 36 * _CO1), lambda i: (0, 0)),
            pl.BlockSpec((_CO2R, 1), lambda i: (0, 0)),
        ],
        out_specs=pl.BlockSpec((img, _CO2R, _ND2), lambda i: (i, 0, 0)),
        compiler_params=pltpu.CompilerParams(
            dimension_semantics=("parallel",)),
    )(xq, w1s, b1, w2s, b2)


def _fc_stage(x, w1, b1, w2, b2, bt):
    b, d = x.shape
    h1 = w1.shape[1]
    h2 = w2.shape[1]
    return pl.pallas_call(
        _fc_body,
        out_shape=jax.ShapeDtypeStruct((b, h2), jnp.float32),
        grid=(b // bt,),
        in_specs=[
            pl.BlockSpec((bt, d), lambda i: (i, 0)),
            pl.BlockSpec((d, h1), lambda i: (0, 0)),
            pl.BlockSpec((1, h1), lambda i: (0, 0)),
            pl.BlockSpec((h1, h2), lambda i: (0, 0)),
            pl.BlockSpec((1, h2), lambda i: (0, 0)),
        ],
        out_specs=pl.BlockSpec((bt, h2), lambda i: (i, 0)),
        compiler_params=pltpu.CompilerParams(
            dimension_semantics=("parallel",)),
    )(x, w1, b1, w2, b2)


def kernel(conv1_w, conv1_b, conv2_w, conv2_b, fc1_w, fc1_b, fc2_w, fc2_b, x):
    bsz, cin = x.shape[0], x.shape[1]
    img = 32 if bsz % 32 == 0 else 1
    bt = 128 if bsz % 128 == 0 else bsz

    w1s = _build_w1(conv1_w, cin)
    w2s = _build_w2(conv2_w)
    b1 = jnp.pad(conv1_b, ((0, _CO1 - conv1_b.shape[0]), (0, 0)))

    nchunk = 4 if bsz % (4 * img) == 0 else 1
    bc = bsz // nchunk
    outs = []
    for k in range(nchunk):
        xk = x[k * bc:(k + 1) * bc]
        xq = _phase16_split(xk)
        y2 = _conv_tower(xq, w1s, b1, w2s, conv2_b, cin=cin, img=img)
        y2 = y2.reshape(bc, _CO2R * _ND2)
        outs.append(_fc_stage(y2, fc1_w.astype(jnp.bfloat16), fc1_b,
                              fc2_w, fc2_b, min(bt, bc)))
    return jnp.concatenate(outs, axis=0)


# speedup vs baseline: 1.0296x; 1.0296x over previous
import functools

import numpy as np

import jax
import jax.numpy as jnp
from jax.experimental import pallas as pl
from jax.experimental.pallas import tpu as pltpu

_K = 5
_CO1 = 16
_CO2 = 24
_CO2R = 20
_WQ = 21
_ND1 = 21 * _WQ
_ND2 = 18 * _WQ
_L16 = 23 * _WQ


def _phase16_split(x):
    b, c, h, w = x.shape
    hq, wq = h // 4, w // 4
    t = x.astype(jnp.bfloat16)
    t = t.reshape(b, c, hq, 4, wq, 4).transpose(0, 3, 5, 1, 2, 4)
    t = t.reshape(b, 16 * c, hq, wq)
    t = jnp.pad(t, ((0, 0), (0, 0), (0, 2), (0, 0)))
    return t.reshape(b, 16 * c, (hq + 2) * wq)


@functools.lru_cache(maxsize=None)
def _placement1():
    t = np.zeros((4, 4, _K * _K, 64), np.float32)
    for p in (0, 1):
        for q in (0, 1):
            for po in range(4):
                pa, pb = po // 2, po % 2
                for kh in range(_K):
                    for kw in range(_K):
                        dr, rm = divmod(2 * p + pa + kh, 4)
                        dc, cm = divmod(2 * q + pb + kw, 4)
                        slot = (dr * 2 + dc) * 16 + rm * 4 + cm
                        t[p * 2 + q, po, kh * _K + kw, slot] = 1.0
    return t


@functools.lru_cache(maxsize=None)
def _placement2():
    t = np.zeros((4, _K * _K, 36), np.float32)
    for po in range(4):
        pa, pb = po // 2, po % 2
        for kh in range(_K):
            for kw in range(_K):
                dr, p = divmod(pa + kh, 2)
                dc, q = divmod(pb + kw, 2)
                t[po, kh * _K + kw, (dr * 3 + dc) * 4 + p * 2 + q] = 1.0
    return t


def _build_w1(conv1_w, cin):
    ws = jnp.einsum("toi,qpts->qposi", conv1_w, jnp.asarray(_placement1()))
    ws = jnp.pad(ws, ((0, 0), (0, 0), (0, _CO1 - conv1_w.shape[1]),
                      (0, 0), (0, 0)))
    return ws.reshape(4 * 4 * _CO1, 64 * cin).astype(jnp.bfloat16)


def _build_w2(conv2_w):
    ws = jnp.einsum("toi,pts->posi", conv2_w, jnp.asarray(_placement2()))
    ws = jnp.pad(ws, ((0, 0), (0, _CO2 - conv2_w.shape[1]),
                      (0, 0), (0, _CO1 - conv2_w.shape[2])))
    return ws.reshape(4 * _CO2, 36 * _CO1).astype(jnp.bfloat16)


def _conv_tower_body(xq_ref, w1_ref, b1_ref, w2_ref, b2_ref, o_ref, *,
                     cin, img):
    w1 = w1_ref[...]
    b1 = b1_ref[...]
    w2 = w2_ref[...]
    b2 = b2_ref[...]
    y1cats = []
    for i in range(img):
        xcat = jnp.concatenate(
            [xq_ref[i, :, off:off + _ND1] for off in (0, 1, _WQ, _WQ + 1)],
            axis=0)
        acc = jnp.dot(w1, xcat, preferred_element_type=jnp.float32)
        y1 = []
        for pq in range(4):
            a = acc[pq * 4 * _CO1:(pq + 1) * 4 * _CO1]
            pooled = jnp.maximum(
                jnp.maximum(a[:_CO1], a[_CO1:2 * _CO1]),
                jnp.maximum(a[2 * _CO1:3 * _CO1], a[3 * _CO1:]))
            y1.append(jnp.maximum(pooled + b1, 0.0)
                      .astype(jnp.bfloat16))
        y1cats.append(jnp.concatenate(y1, axis=0))
    for i in range(img):
        xcat2 = jnp.concatenate(
            [y1cats[i][:, dr * _WQ + dc:dr * _WQ + dc + _ND2]
             for dr in range(3) for dc in range(3)],
            axis=0)
        acc2 = jnp.dot(w2, xcat2, preferred_element_type=jnp.float32)
        pooled = jnp.maximum(
            jnp.maximum(acc2[:_CO2], acc2[_CO2:2 * _CO2]),
            jnp.maximum(acc2[2 * _CO2:3 * _CO2], acc2[3 * _CO2:]))
        o_ref[i] = jnp.maximum(pooled[:_CO2R] + b2, 0.0).astype(jnp.bfloat16)


def _fc_body(x_ref, w1_ref, b1_ref, w2_ref, b2_ref, o_ref):
    h = jnp.dot(x_ref[...], w1_ref[...], preferred_element_type=jnp.float32)
    h = jnp.maximum(h + b1_ref[...], 0.0)
    z = jnp.dot(h, w2_ref[...], preferred_element_type=jnp.float32)
    z = z + b2_ref[...]
    z = z - jnp.max(z, axis=-1, keepdims=True)
    o_ref[...] = z - jnp.log(jnp.sum(jnp.exp(z), axis=-1, keepdims=True))


def _conv_tower(xq, w1s, b1, w2s, b2, *, cin, img):
    b = xq.shape[0]
    body = functools.partial(_conv_tower_body, cin=cin, img=img)
    return pl.pallas_call(
        body,
        out_shape=jax.ShapeDtypeStruct((b, _CO2R, _ND2), jnp.bfloat16),
        grid=(b // img,),
        in_specs=[
            pl.BlockSpec((img, 16 * cin, _L16), lambda i: (i, 0, 0)),
            pl.BlockSpec((4 * 4 * _CO1, 64 * cin), lambda i: (0, 0)),
            pl.BlockSpec((_CO1, 1), lambda i: (0, 0)),
            pl.BlockSpec((4 * _CO2, 36 * _CO1), lambda i: (0, 0)),
            pl.BlockSpec((_CO2R, 1), lambda i: (0, 0)),
        ],
        out_specs=pl.BlockSpec((img, _CO2R, _ND2), lambda i: (i, 0, 0)),
        compiler_params=pltpu.CompilerParams(
            dimension_semantics=("parallel",)),
    )(xq, w1s, b1, w2s, b2)


def _fc_stage(x, w1, b1, w2, b2, bt):
    b, d = x.shape
    h1 = w1.shape[1]
    h2 = w2.shape[1]
    return pl.pallas_call(
        _fc_body,
        out_shape=jax.ShapeDtypeStruct((b, h2), jnp.float32),
        grid=(b // bt,),
        in_specs=[
            pl.BlockSpec((bt, d), lambda i: (i, 0)),
            pl.BlockSpec((d, h1), lambda i: (0, 0)),
            pl.BlockSpec((1, h1), lambda i: (0, 0)),
            pl.BlockSpec((h1, h2), lambda i: (0, 0)),
            pl.BlockSpec((1, h2), lambda i: (0, 0)),
        ],
        out_specs=pl.BlockSpec((bt, h2), lambda i: (i, 0)),
        compiler_params=pltpu.CompilerParams(
            dimension_semantics=("parallel",)),
    )(x, w1, b1, w2, b2)


def kernel(conv1_w, conv1_b, conv2_w, conv2_b, fc1_w, fc1_b, fc2_w, fc2_b, x):
    bsz, cin = x.shape[0], x.shape[1]
    img = 32 if bsz % 32 == 0 else 1
    bt = 128 if bsz % 128 == 0 else bsz

    w1s = _build_w1(conv1_w, cin)
    w2s = _build_w2(conv2_w)
    b1 = jnp.pad(conv1_b, ((0, _CO1 - conv1_b.shape[0]), (0, 0)))

    nchunk = 4 if bsz % (4 * img) == 0 else 1
    bc = bsz // nchunk
    outs = []
    for k in range(nchunk):
        xk = x[k * bc:(k + 1) * bc]
        xq = _phase16_split(xk)
        y2 = _conv_tower(xq, w1s, b1, w2s, conv2_b, cin=cin, img=img)
        y2 = y2.reshape(bc, _CO2R * _ND2)
        outs.append(_fc_stage(y2, fc1_w.astype(jnp.bfloat16), fc1_b,
                              fc2_w, fc2_b, min(bt, bc)))
    return jnp.concatenate(outs, axis=0)
